# Initial kernel scaffold; baseline (speedup 1.0000x reference)
import jax
import jax.numpy as jnp
from jax.experimental import pallas as pl


def kernel(x, edge_index, node_type, edge_type, k_w, q_w, v_w, a_w, w_att, w_msg, mu, skip):
    raise NotImplementedError("write your pallas kernel here")


# speedup vs baseline: 67.5885x; 67.5885x over previous
import functools
import math

import jax
import jax.numpy as jnp
from jax import lax
from jax.experimental import pallas as pl
from jax.experimental.pallas import tpu as pltpu
from jax.experimental.pallas import tpu_sc as plsc

N = 10000
E = 320000
DIN = 128
DOUT = 128
H = 8
DK = 16
NT = 3
ET = 4

BN = 1000
NBLK = N // BN
CH = 80
NCHUNK = (E // 32) // CH
EPW = E // 32
ROWS_A = 640
ROWS_B = 624
ZR = 104


def _proj_body(x_ref, oh_ref, kw_ref, qw_ref, vw_ref, wa_ref, wm_ref,
               krel_ref, vrel_ref, q_ref):
    x_b = x_ref[...]
    oh = oh_ref[...]
    k_b = jnp.zeros((BN, DOUT), jnp.float32)
    v_b = jnp.zeros((BN, DOUT), jnp.float32)
    q_b = jnp.zeros((BN, DOUT), jnp.float32)
    for t in range(NT):
        sel = oh[:, t][:, None]
        k_b = k_b + jnp.dot(x_b, kw_ref[t], preferred_element_type=jnp.float32) * sel
        v_b = v_b + jnp.dot(x_b, vw_ref[t], preferred_element_type=jnp.float32) * sel
        q_b = q_b + jnp.dot(x_b, qw_ref[t], preferred_element_type=jnp.float32) * sel
    q_ref[...] = q_b
    krel_ref[...] = jnp.dot(k_b, wa_ref[0], preferred_element_type=jnp.float32)
    vrel_ref[...] = jnp.dot(v_b, wm_ref[0], preferred_element_type=jnp.float32)


def _project(x, oh8, k_w, q_w, v_w, wa_blk, wm_blk):
    grid = (NBLK, ET)
    node_spec = pl.BlockSpec((BN, DIN), lambda i, e: (i, 0))
    oh_spec = pl.BlockSpec((BN, 8), lambda i, e: (i, 0))
    w_spec = pl.BlockSpec((NT, DIN, DOUT), lambda i, e: (0, 0, 0))
    rel_spec = pl.BlockSpec((1, DOUT, DOUT), lambda i, e: (e, 0, 0))
    out_rel_spec = pl.BlockSpec((BN, DOUT), lambda i, e: (e * NBLK + i, 0))
    return pl.pallas_call(
        _proj_body,
        grid=grid,
        in_specs=[node_spec, oh_spec, w_spec, w_spec, w_spec, rel_spec, rel_spec],
        out_specs=[out_rel_spec, out_rel_spec, node_spec],
        out_shape=[
            jax.ShapeDtypeStruct((ET * N, DOUT), jnp.float32),
            jax.ShapeDtypeStruct((ET * N, DOUT), jnp.float32),
            jax.ShapeDtypeStruct((N, DOUT), jnp.float32),
        ],
    )(x, oh8, k_w, q_w, v_w, wa_blk, wm_blk)


_GD = lax.GatherDimensionNumbers(
    offset_dims=(), collapsed_slice_dims=(0,), start_index_map=(0,))


def _lane_bcast(v, lane):
    idx = jnp.full((16, 1), lane, jnp.int32)
    return lax.gather(v, idx, _GD, slice_sizes=(1,),
                      mode=lax.GatherScatterMode.PROMISE_IN_BOUNDS)


def _perm(v, idx):
    return lax.gather(v, idx[:, None], _GD, slice_sizes=(1,),
                      mode=lax.GatherScatterMode.PROMISE_IN_BOUNDS)


def _sum_bcast(v, lanes):
    for m in (8, 4, 2, 1):
        v = v + _perm(v, lanes ^ m)
    return v
def _edge_body(krel_hbm, q_hbm, vrel_hbm, src_hbm, dst_hbm, et_hbm,
               accp_hbm, exv_hbm,
               srcv, dstv, etv, idxv, krows, qrows, vrows,
               exrow, idxf, idx64, semk, semq, semv, shacc):
    cid = lax.axis_index("c")
    sid = lax.axis_index("s")
    wid = cid * 16 + sid
    lanes = lax.iota(jnp.int32, 16)

    def _fill_iota(ref, base, length):
        for o in range(0, length, 16):
            ref[pl.ds(o, 16)] = base + o + lanes

    rbase = jnp.where(sid == 0, 0, ROWS_A + (sid - 1) * ROWS_B)

    z16 = jnp.zeros((16,), jnp.float32)

    def _zero_krows(i, _):
        for j in range(DOUT // 16):
            krows[i, pl.ds(j * 16, 16)] = z16
        return 0
    lax.fori_loop(0, CH, _zero_krows, 0)

    for r in range(7):
        _fill_iota(idxf, rbase + r * CH, CH)
        pltpu.sync_copy(krows, shacc.at[idxf])

    @pl.when(sid == 0)
    def _():
        _fill_iota(idxf, 7 * CH, CH)
        pltpu.sync_copy(krows, shacc.at[idxf])

    @pl.when(sid > 0)
    def _():
        _fill_iota(idx64, rbase + 7 * CH, 64)
        pltpu.sync_copy(krows.at[pl.ds(0, 64)], shacc.at[idx64])
    plsc.subcore_barrier()

    ebase = wid * EPW

    def _chunk(c, _):
        base = ebase + c * CH
        pltpu.sync_copy(src_hbm.at[pl.ds(base, CH)], srcv)
        pltpu.sync_copy(dst_hbm.at[pl.ds(base, CH)], dstv)
        pltpu.sync_copy(et_hbm.at[pl.ds(base, CH)], etv)
        for i in range(CH // 16):
            s16 = srcv[pl.ds(i * 16, 16)]
            e16 = etv[pl.ds(i * 16, 16)]
            idxv[pl.ds(i * 16, 16)] = e16 * N + s16
        ck = pltpu.async_copy(krel_hbm.at[idxv], krows, semk)
        cq = pltpu.async_copy(q_hbm.at[dstv], qrows, semq)
        cv = pltpu.async_copy(vrel_hbm.at[idxv], vrows, semv)
        ck.wait()
        cq.wait()
        cv.wait()

        def _pair(p, _):
            e0 = p * 2
            e1 = e0 + 1
            sc = jnp.zeros((16,), jnp.float32)
            for h in range(H):
                pk = krows[e0, pl.ds(h * DK, DK)] * qrows[e0, pl.ds(h * DK, DK)]
                sc = jnp.where(lanes == h, _sum_bcast(pk, lanes), sc)
                pk1 = krows[e1, pl.ds(h * DK, DK)] * qrows[e1, pl.ds(h * DK, DK)]
                sc = jnp.where(lanes == 15 - h, _sum_bcast(pk1, lanes), sc)
            exv = jnp.exp(jnp.clip(sc, -60.0, 60.0))
            exrow[e0, :] = jnp.where(lanes < 8, exv, 0.0)
            exrow[e1, :] = jnp.where(lanes < 8, lax.rev(exv, (0,)), 0.0)
            for h in range(H):
                b0 = _lane_bcast(exv, h)
                vrows[e0, pl.ds(h * DK, DK)] = vrows[e0, pl.ds(h * DK, DK)] * b0
                b1 = _lane_bcast(exv, 15 - h)
                vrows[e1, pl.ds(h * DK, DK)] = vrows[e1, pl.ds(h * DK, DK)] * b1
            return 0
        lax.fori_loop(0, CH // 2, _pair, 0)

        pltpu.sync_copy(vrows, shacc.at[dstv], add=True)
        pltpu.sync_copy(exrow, exv_hbm.at[pl.ds(base, CH)])
        return 0

    lax.fori_loop(0, NCHUNK, _chunk, 0)
    plsc.subcore_barrier()

    for r in range(7):
        fb = pl.multiple_of(rbase + r * CH, 8)
        _fill_iota(idxf, fb, CH)
        pltpu.sync_copy(shacc.at[idxf], krows)
        pltpu.sync_copy(krows, accp_hbm.at[cid].at[pl.ds(fb, CH)])

    @pl.when(sid == 0)
    def _():
        _fill_iota(idxf, 7 * CH, CH)
        pltpu.sync_copy(shacc.at[idxf], krows)
        pltpu.sync_copy(krows, accp_hbm.at[cid].at[pl.ds(7 * CH, CH)])

    @pl.when(sid > 0)
    def _():
        fb = pl.multiple_of(rbase + 7 * CH, 8)
        _fill_iota(idx64, fb, 64)
        pltpu.sync_copy(shacc.at[idx64], krows.at[pl.ds(0, 64)])
        pltpu.sync_copy(krows.at[pl.ds(0, 64)], accp_hbm.at[cid].at[pl.ds(fb, 64)])


def _edge_pass(krel, q, vrel, src, dst, et):
    mesh = plsc.VectorSubcoreMesh(core_axis_name="c", subcore_axis_name="s")
    fn = pl.kernel(
        _edge_body,
        out_type=[
            jax.ShapeDtypeStruct((2, N, DOUT), jnp.float32),
            jax.ShapeDtypeStruct((E, 16), jnp.float32),
        ],
        mesh=mesh,
        scratch_types=[
            pltpu.VMEM((CH,), jnp.int32),
            pltpu.VMEM((CH,), jnp.int32),
            pltpu.VMEM((CH,), jnp.int32),
            pltpu.VMEM((CH,), jnp.int32),
            pltpu.VMEM((CH, DOUT), jnp.float32),
            pltpu.VMEM((CH, DOUT), jnp.float32),
            pltpu.VMEM((CH, DOUT), jnp.float32),
            pltpu.VMEM((CH, 16), jnp.float32),
            pltpu.VMEM((CH,), jnp.int32),
            pltpu.VMEM((64,), jnp.int32),
            pltpu.SemaphoreType.DMA,
            pltpu.SemaphoreType.DMA,
            pltpu.SemaphoreType.DMA,
            pltpu.VMEM_SHARED((N, DOUT), jnp.float32),
        ],
    )
    return fn(krel, q, vrel, src, dst, et)


def _ex_body(dst_hbm, exv_hbm, exp2_hbm,
             dstv, exrow, febuf, idxf, idx64, shex):
    cid = lax.axis_index("c")
    sid = lax.axis_index("s")
    wid = cid * 16 + sid
    lanes = lax.iota(jnp.int32, 16)

    def _fill_iota(ref, base, length):
        for o in range(0, length, 16):
            ref[pl.ds(o, 16)] = base + o + lanes

    rbase = jnp.where(sid == 0, 0, ROWS_A + (sid - 1) * ROWS_B)

    z16 = jnp.zeros((16,), jnp.float32)

    def _zero_feb(i, _):
        febuf[i, :] = z16
        return 0
    lax.fori_loop(0, CH, _zero_feb, 0)

    for r in range(7):
        _fill_iota(idxf, rbase + r * CH, CH)
        pltpu.sync_copy(febuf, shex.at[idxf])

    @pl.when(sid == 0)
    def _():
        _fill_iota(idxf, 7 * CH, CH)
        pltpu.sync_copy(febuf, shex.at[idxf])

    @pl.when(sid > 0)
    def _():
        _fill_iota(idx64, rbase + 7 * CH, 64)
        pltpu.sync_copy(febuf.at[pl.ds(0, 64)], shex.at[idx64])
    plsc.subcore_barrier()

    ebase = wid * EPW

    def _chunk(c, _):
        base = ebase + c * CH
        pltpu.sync_copy(dst_hbm.at[pl.ds(base, CH)], dstv)
        pltpu.sync_copy(exv_hbm.at[pl.ds(base, CH)], exrow)
        pltpu.sync_copy(exrow, shex.at[dstv], add=True)
        return 0

    lax.fori_loop(0, NCHUNK, _chunk, 0)
    plsc.subcore_barrier()

    for r in range(7):
        fb = pl.multiple_of(rbase + r * CH, 8)
        _fill_iota(idxf, fb, CH)
        pltpu.sync_copy(shex.at[idxf], febuf)
        pltpu.sync_copy(febuf, exp2_hbm.at[cid].at[pl.ds(fb, CH)])

    @pl.when(sid == 0)
    def _():
        _fill_iota(idxf, 7 * CH, CH)
        pltpu.sync_copy(shex.at[idxf], febuf)
        pltpu.sync_copy(febuf, exp2_hbm.at[cid].at[pl.ds(7 * CH, CH)])

    @pl.when(sid > 0)
    def _():
        fb = pl.multiple_of(rbase + 7 * CH, 8)
        _fill_iota(idx64, fb, 64)
        pltpu.sync_copy(shex.at[idx64], febuf.at[pl.ds(0, 64)])
        pltpu.sync_copy(febuf.at[pl.ds(0, 64)],
                        exp2_hbm.at[cid].at[pl.ds(fb, 64)])


def _ex_pass(dst, exvals):
    mesh = plsc.VectorSubcoreMesh(core_axis_name="c", subcore_axis_name="s")
    fn = pl.kernel(
        _ex_body,
        out_type=[jax.ShapeDtypeStruct((2, N, 16), jnp.float32)],
        mesh=mesh,
        scratch_types=[
            pltpu.VMEM((CH,), jnp.int32),
            pltpu.VMEM((CH, 16), jnp.float32),
            pltpu.VMEM((CH, 16), jnp.float32),
            pltpu.VMEM((CH,), jnp.int32),
            pltpu.VMEM((64,), jnp.int32),
            pltpu.VMEM_SHARED((N, 16), jnp.float32),
        ],
    )
    return fn(dst, exvals)[0]


def _epi_body(accp_ref, exs_ref, x_ref, oh_ref, oha_ref, aw_ref, out_ref):
    acc = accp_ref[0] + accp_ref[1]
    ssum = exs_ref[0] + exs_ref[1]
    row = lax.broadcasted_iota(jnp.int32, (16, DOUT), 0)
    col = lax.broadcasted_iota(jnp.int32, (16, DOUT), 1)
    rep = jnp.where(col // DK == row, 1.0, 0.0).astype(jnp.float32)
    den = jnp.dot(ssum, rep, preferred_element_type=jnp.float32) + 1e-9
    agg = acc / den
    t = jax.nn.gelu(agg)
    oh = oh_ref[...]
    out = jnp.zeros((BN, DOUT), jnp.float32)
    for tt in range(NT):
        out = out + jnp.dot(t, aw_ref[tt], preferred_element_type=jnp.float32) * oh[:, tt][:, None]
    alpha = jnp.sum(oha_ref[...], axis=1, keepdims=True)
    out_ref[...] = alpha * out + (1.0 - alpha) * x_ref[...]


def _epilogue(accp, exs, x, oh8, oha, a_w):
    grid = (NBLK,)
    return pl.pallas_call(
        _epi_body,
        grid=grid,
        in_specs=[
            pl.BlockSpec((2, BN, DOUT), lambda i: (0, i, 0)),
            pl.BlockSpec((2, BN, 16), lambda i: (0, i, 0)),
            pl.BlockSpec((BN, DIN), lambda i: (i, 0)),
            pl.BlockSpec((BN, 8), lambda i: (i, 0)),
            pl.BlockSpec((BN, 8), lambda i: (i, 0)),
            pl.BlockSpec((NT, DOUT, DOUT), lambda i: (0, 0, 0)),
        ],
        out_specs=pl.BlockSpec((BN, DOUT), lambda i: (i, 0)),
        out_shape=jax.ShapeDtypeStruct((N, DOUT), jnp.float32),
    )(accp, exs, x, oh8, oha, a_w)


def kernel(x, edge_index, node_type, edge_type, k_w, q_w, v_w, a_w,
           w_att, w_msg, mu, skip):
    src = edge_index[0]
    dst = edge_index[1]

    oh = (node_type[:, None] == jnp.arange(NT, dtype=jnp.int32)[None, :])
    oh8 = jnp.pad(oh.astype(jnp.float32), ((0, 0), (0, 8 - NT)))
    alpha_t = jax.nn.sigmoid(skip)
    oha = oh8 * jnp.pad(alpha_t, (0, 8 - NT))[None, :]

    eye = jnp.eye(H, dtype=jnp.float32)
    wa = w_att * (mu / math.sqrt(DK))[:, :, None, None]
    wa_blk = jnp.einsum('thij,hg->thigj', wa, eye).reshape(ET, DOUT, DOUT)
    wm_blk = jnp.einsum('thij,hg->thigj', w_msg, eye).reshape(ET, DOUT, DOUT)

    krel, vrel, q = _project(x, oh8, k_w, q_w, v_w, wa_blk, wm_blk)
    accp, exvals = _edge_pass(krel, q, vrel, src, dst, edge_type)
    exp2 = _ex_pass(dst, exvals)
    return _epilogue(accp, exp2, x, oh8, oha, a_w)


# speedup vs baseline: 82.3633x; 1.2186x over previous
import functools
import math

import jax
import jax.numpy as jnp
from jax import lax
from jax.experimental import pallas as pl
from jax.experimental.pallas import tpu as pltpu
from jax.experimental.pallas import tpu_sc as plsc

N = 10000
E = 320000
DIN = 128
DOUT = 128
H = 8
DK = 16
NT = 3
ET = 4

BN = 1000
NBLK = N // BN
CH = 80
NCHUNK = (E // 32) // CH
EPW = E // 32
ROWS_A = 640
ROWS_B = 624
ZR = 104


def _proj_body(x_ref, oh_ref, kw_ref, qw_ref, vw_ref, wa_ref, wm_ref,
               krel_ref, vrel_ref, q_ref):
    x_b = x_ref[...]
    oh = oh_ref[...]
    k_b = jnp.zeros((BN, DOUT), jnp.float32)
    v_b = jnp.zeros((BN, DOUT), jnp.float32)
    q_b = jnp.zeros((BN, DOUT), jnp.float32)
    for t in range(NT):
        sel = oh[:, t][:, None]
        k_b = k_b + jnp.dot(x_b, kw_ref[t], preferred_element_type=jnp.float32) * sel
        v_b = v_b + jnp.dot(x_b, vw_ref[t], preferred_element_type=jnp.float32) * sel
        q_b = q_b + jnp.dot(x_b, qw_ref[t], preferred_element_type=jnp.float32) * sel
    q_ref[...] = q_b
    krel_ref[...] = jnp.dot(k_b, wa_ref[0], preferred_element_type=jnp.float32)
    vrel_ref[...] = jnp.dot(v_b, wm_ref[0], preferred_element_type=jnp.float32)


def _project(x, oh8, k_w, q_w, v_w, wa_blk, wm_blk):
    grid = (NBLK, ET)
    node_spec = pl.BlockSpec((BN, DIN), lambda i, e: (i, 0))
    oh_spec = pl.BlockSpec((BN, 8), lambda i, e: (i, 0))
    w_spec = pl.BlockSpec((NT, DIN, DOUT), lambda i, e: (0, 0, 0))
    rel_spec = pl.BlockSpec((1, DOUT, DOUT), lambda i, e: (e, 0, 0))
    out_rel_spec = pl.BlockSpec((BN, DOUT), lambda i, e: (e * NBLK + i, 0))
    return pl.pallas_call(
        _proj_body,
        grid=grid,
        in_specs=[node_spec, oh_spec, w_spec, w_spec, w_spec, rel_spec, rel_spec],
        out_specs=[out_rel_spec, out_rel_spec, node_spec],
        out_shape=[
            jax.ShapeDtypeStruct((ET * N, DOUT), jnp.float32),
            jax.ShapeDtypeStruct((ET * N, DOUT), jnp.float32),
            jax.ShapeDtypeStruct((N, DOUT), jnp.float32),
        ],
    )(x, oh8, k_w, q_w, v_w, wa_blk, wm_blk)


_GD = lax.GatherDimensionNumbers(
    offset_dims=(), collapsed_slice_dims=(0,), start_index_map=(0,))


def _lane_bcast(v, lane):
    idx = jnp.full((16, 1), lane, jnp.int32)
    return lax.gather(v, idx, _GD, slice_sizes=(1,),
                      mode=lax.GatherScatterMode.PROMISE_IN_BOUNDS)


def _perm(v, idx):
    return lax.gather(v, idx[:, None], _GD, slice_sizes=(1,),
                      mode=lax.GatherScatterMode.PROMISE_IN_BOUNDS)


def _sum_bcast(v, lanes):
    for m in (8, 4, 2, 1):
        v = v + _perm(v, lanes ^ m)
    return v
def _edge_body(krel_hbm, q_hbm, vrel_hbm, src_hbm, dst_hbm, et_hbm,
               accp_hbm, exv_hbm,
               srcv, dstv, etv, idxv, krows, qrows, vrows,
               exrow, idxf, idx64, semk, semq, semv, shacc):
    cid = lax.axis_index("c")
    sid = lax.axis_index("s")
    wid = cid * 16 + sid
    lanes = lax.iota(jnp.int32, 16)

    def _fill_iota(ref, base, length):
        for o in range(0, length, 16):
            ref[pl.ds(o, 16)] = base + o + lanes

    rbase = jnp.where(sid == 0, 0, ROWS_A + (sid - 1) * ROWS_B)

    z16 = jnp.zeros((16,), jnp.float32)

    def _zero_krows(i, _):
        for j in range(DOUT // 16):
            krows[i, pl.ds(j * 16, 16)] = z16
        return 0
    lax.fori_loop(0, CH, _zero_krows, 0)

    for r in range(7):
        _fill_iota(idxf, rbase + r * CH, CH)
        pltpu.sync_copy(krows, shacc.at[idxf])

    @pl.when(sid == 0)
    def _():
        _fill_iota(idxf, 7 * CH, CH)
        pltpu.sync_copy(krows, shacc.at[idxf])

    @pl.when(sid > 0)
    def _():
        _fill_iota(idx64, rbase + 7 * CH, 64)
        pltpu.sync_copy(krows.at[pl.ds(0, 64)], shacc.at[idx64])
    plsc.subcore_barrier()

    ebase = wid * EPW

    def _chunk(c, _):
        base = ebase + c * CH
        c1 = pltpu.async_copy(src_hbm.at[pl.ds(base, CH)], srcv, semk)
        c2 = pltpu.async_copy(dst_hbm.at[pl.ds(base, CH)], dstv, semq)
        c3 = pltpu.async_copy(et_hbm.at[pl.ds(base, CH)], etv, semv)
        c1.wait()
        c2.wait()
        c3.wait()
        for i in range(CH // 16):
            s16 = srcv[pl.ds(i * 16, 16)]
            e16 = etv[pl.ds(i * 16, 16)]
            idxv[pl.ds(i * 16, 16)] = e16 * N + s16
        ck = pltpu.async_copy(krel_hbm.at[idxv], krows, semk)
        cq = pltpu.async_copy(q_hbm.at[dstv], qrows, semq)
        cv = pltpu.async_copy(vrel_hbm.at[idxv], vrows, semv)
        ck.wait()
        cq.wait()
        cv.wait()

        def _pair(p, _):
            e0 = p * 2
            e1 = e0 + 1
            sc = jnp.zeros((16,), jnp.float32)
            for h in range(H):
                pk = krows[e0, pl.ds(h * DK, DK)] * qrows[e0, pl.ds(h * DK, DK)]
                sc = jnp.where(lanes == h, _sum_bcast(pk, lanes), sc)
                pk1 = krows[e1, pl.ds(h * DK, DK)] * qrows[e1, pl.ds(h * DK, DK)]
                sc = jnp.where(lanes == 15 - h, _sum_bcast(pk1, lanes), sc)
            exv = jnp.exp(jnp.clip(sc, -60.0, 60.0))
            exrow[e0, :] = jnp.where(lanes < 8, exv, 0.0)
            exrow[e1, :] = jnp.where(lanes < 8, lax.rev(exv, (0,)), 0.0)
            for h in range(H):
                b0 = _lane_bcast(exv, h)
                vrows[e0, pl.ds(h * DK, DK)] = vrows[e0, pl.ds(h * DK, DK)] * b0
                b1 = _lane_bcast(exv, 15 - h)
                vrows[e1, pl.ds(h * DK, DK)] = vrows[e1, pl.ds(h * DK, DK)] * b1
            return 0
        lax.fori_loop(0, CH // 2, _pair, 0)

        s1 = pltpu.async_copy(vrows, shacc.at[dstv], semk, add=True)
        s2 = pltpu.async_copy(exrow, exv_hbm.at[pl.ds(base, CH)], semq)
        s1.wait()
        s2.wait()
        return 0

    lax.fori_loop(0, NCHUNK, _chunk, 0)
    plsc.subcore_barrier()

    for r in range(7):
        fb = pl.multiple_of(rbase + r * CH, 8)
        _fill_iota(idxf, fb, CH)
        pltpu.sync_copy(shacc.at[idxf], krows)
        pltpu.sync_copy(krows, accp_hbm.at[cid].at[pl.ds(fb, CH)])

    @pl.when(sid == 0)
    def _():
        _fill_iota(idxf, 7 * CH, CH)
        pltpu.sync_copy(shacc.at[idxf], krows)
        pltpu.sync_copy(krows, accp_hbm.at[cid].at[pl.ds(7 * CH, CH)])

    @pl.when(sid > 0)
    def _():
        fb = pl.multiple_of(rbase + 7 * CH, 8)
        _fill_iota(idx64, fb, 64)
        pltpu.sync_copy(shacc.at[idx64], krows.at[pl.ds(0, 64)])
        pltpu.sync_copy(krows.at[pl.ds(0, 64)], accp_hbm.at[cid].at[pl.ds(fb, 64)])


def _edge_pass(krel, q, vrel, src, dst, et):
    mesh = plsc.VectorSubcoreMesh(core_axis_name="c", subcore_axis_name="s")
    fn = pl.kernel(
        _edge_body,
        out_type=[
            jax.ShapeDtypeStruct((2, N, DOUT), jnp.float32),
            jax.ShapeDtypeStruct((E, 16), jnp.float32),
        ],
        mesh=mesh,
        scratch_types=[
            pltpu.VMEM((CH,), jnp.int32),
            pltpu.VMEM((CH,), jnp.int32),
            pltpu.VMEM((CH,), jnp.int32),
            pltpu.VMEM((CH,), jnp.int32),
            pltpu.VMEM((CH, DOUT), jnp.float32),
            pltpu.VMEM((CH, DOUT), jnp.float32),
            pltpu.VMEM((CH, DOUT), jnp.float32),
            pltpu.VMEM((CH, 16), jnp.float32),
            pltpu.VMEM((CH,), jnp.int32),
            pltpu.VMEM((64,), jnp.int32),
            pltpu.SemaphoreType.DMA,
            pltpu.SemaphoreType.DMA,
            pltpu.SemaphoreType.DMA,
            pltpu.VMEM_SHARED((N, DOUT), jnp.float32),
        ],
    )
    return fn(krel, q, vrel, src, dst, et)


def _ex_body(dst_hbm, exv_hbm, exp2_hbm,
             dstv, exrow, febuf, idxf, idx64, sema, semb, shex):
    cid = lax.axis_index("c")
    sid = lax.axis_index("s")
    wid = cid * 16 + sid
    lanes = lax.iota(jnp.int32, 16)

    def _fill_iota(ref, base, length):
        for o in range(0, length, 16):
            ref[pl.ds(o, 16)] = base + o + lanes

    rbase = jnp.where(sid == 0, 0, ROWS_A + (sid - 1) * ROWS_B)

    z16 = jnp.zeros((16,), jnp.float32)

    def _zero_feb(i, _):
        febuf[i, :] = z16
        return 0
    lax.fori_loop(0, CH, _zero_feb, 0)

    for r in range(7):
        _fill_iota(idxf, rbase + r * CH, CH)
        pltpu.sync_copy(febuf, shex.at[idxf])

    @pl.when(sid == 0)
    def _():
        _fill_iota(idxf, 7 * CH, CH)
        pltpu.sync_copy(febuf, shex.at[idxf])

    @pl.when(sid > 0)
    def _():
        _fill_iota(idx64, rbase + 7 * CH, 64)
        pltpu.sync_copy(febuf.at[pl.ds(0, 64)], shex.at[idx64])
    plsc.subcore_barrier()

    ebase = wid * EPW

    def _chunk(c, _):
        base = ebase + c * CH
        c1 = pltpu.async_copy(dst_hbm.at[pl.ds(base, CH)], dstv, sema)
        c2 = pltpu.async_copy(exv_hbm.at[pl.ds(base, CH)], exrow, semb)
        c1.wait()
        c2.wait()
        pltpu.sync_copy(exrow, shex.at[dstv], add=True)
        return 0

    lax.fori_loop(0, NCHUNK, _chunk, 0)
    plsc.subcore_barrier()

    for r in range(7):
        fb = pl.multiple_of(rbase + r * CH, 8)
        _fill_iota(idxf, fb, CH)
        pltpu.sync_copy(shex.at[idxf], febuf)
        pltpu.sync_copy(febuf, exp2_hbm.at[cid].at[pl.ds(fb, CH)])

    @pl.when(sid == 0)
    def _():
        _fill_iota(idxf, 7 * CH, CH)
        pltpu.sync_copy(shex.at[idxf], febuf)
        pltpu.sync_copy(febuf, exp2_hbm.at[cid].at[pl.ds(7 * CH, CH)])

    @pl.when(sid > 0)
    def _():
        fb = pl.multiple_of(rbase + 7 * CH, 8)
        _fill_iota(idx64, fb, 64)
        pltpu.sync_copy(shex.at[idx64], febuf.at[pl.ds(0, 64)])
        pltpu.sync_copy(febuf.at[pl.ds(0, 64)],
                        exp2_hbm.at[cid].at[pl.ds(fb, 64)])


def _ex_pass(dst, exvals):
    mesh = plsc.VectorSubcoreMesh(core_axis_name="c", subcore_axis_name="s")
    fn = pl.kernel(
        _ex_body,
        out_type=[jax.ShapeDtypeStruct((2, N, 16), jnp.float32)],
        mesh=mesh,
        scratch_types=[
            pltpu.VMEM((CH,), jnp.int32),
            pltpu.VMEM((CH, 16), jnp.float32),
            pltpu.VMEM((CH, 16), jnp.float32),
            pltpu.VMEM((CH,), jnp.int32),
            pltpu.VMEM((64,), jnp.int32),
            pltpu.SemaphoreType.DMA,
            pltpu.SemaphoreType.DMA,
            pltpu.VMEM_SHARED((N, 16), jnp.float32),
        ],
    )
    return fn(dst, exvals)[0]


def _epi_body(accp_ref, exs_ref, x_ref, oh_ref, oha_ref, aw_ref, out_ref):
    acc = accp_ref[0] + accp_ref[1]
    ssum = exs_ref[0] + exs_ref[1]
    row = lax.broadcasted_iota(jnp.int32, (16, DOUT), 0)
    col = lax.broadcasted_iota(jnp.int32, (16, DOUT), 1)
    rep = jnp.where(col // DK == row, 1.0, 0.0).astype(jnp.float32)
    den = jnp.dot(ssum, rep, preferred_element_type=jnp.float32) + 1e-9
    agg = acc / den
    t = jax.nn.gelu(agg)
    oh = oh_ref[...]
    out = jnp.zeros((BN, DOUT), jnp.float32)
    for tt in range(NT):
        out = out + jnp.dot(t, aw_ref[tt], preferred_element_type=jnp.float32) * oh[:, tt][:, None]
    alpha = jnp.sum(oha_ref[...], axis=1, keepdims=True)
    out_ref[...] = alpha * out + (1.0 - alpha) * x_ref[...]


def _epilogue(accp, exs, x, oh8, oha, a_w):
    grid = (NBLK,)
    return pl.pallas_call(
        _epi_body,
        grid=grid,
        in_specs=[
            pl.BlockSpec((2, BN, DOUT), lambda i: (0, i, 0)),
            pl.BlockSpec((2, BN, 16), lambda i: (0, i, 0)),
            pl.BlockSpec((BN, DIN), lambda i: (i, 0)),
            pl.BlockSpec((BN, 8), lambda i: (i, 0)),
            pl.BlockSpec((BN, 8), lambda i: (i, 0)),
            pl.BlockSpec((NT, DOUT, DOUT), lambda i: (0, 0, 0)),
        ],
        out_specs=pl.BlockSpec((BN, DOUT), lambda i: (i, 0)),
        out_shape=jax.ShapeDtypeStruct((N, DOUT), jnp.float32),
    )(accp, exs, x, oh8, oha, a_w)


def kernel(x, edge_index, node_type, edge_type, k_w, q_w, v_w, a_w,
           w_att, w_msg, mu, skip):
    src = edge_index[0]
    dst = edge_index[1]

    oh = (node_type[:, None] == jnp.arange(NT, dtype=jnp.int32)[None, :])
    oh8 = jnp.pad(oh.astype(jnp.float32), ((0, 0), (0, 8 - NT)))
    alpha_t = jax.nn.sigmoid(skip)
    oha = oh8 * jnp.pad(alpha_t, (0, 8 - NT))[None, :]

    eye = jnp.eye(H, dtype=jnp.float32)
    wa = w_att * (mu / math.sqrt(DK))[:, :, None, None]
    wa_blk = jnp.einsum('thij,hg->thigj', wa, eye).reshape(ET, DOUT, DOUT)
    wm_blk = jnp.einsum('thij,hg->thigj', w_msg, eye).reshape(ET, DOUT, DOUT)

    krel, vrel, q = _project(x, oh8, k_w, q_w, v_w, wa_blk, wm_blk)
    accp, exvals = _edge_pass(krel, q, vrel, src, dst, edge_type)
    exp2 = _ex_pass(dst, exvals)
    return _epilogue(accp, exp2, x, oh8, oha, a_w)
